# Initial kernel scaffold; baseline (speedup 1.0000x reference)
import jax
import jax.numpy as jnp
from jax.experimental import pallas as pl


def kernel(keys, values, queries):
    raise NotImplementedError("write your pallas kernel here")


# speedup vs baseline: 2.4067x; 2.4067x over previous
import functools

import jax
import jax.numpy as jnp
from jax.experimental import pallas as pl
from jax.experimental.pallas import tpu as pltpu

HIDDEN_SIZE = 64
TOP_K = 8
EPS = 1e-06


def _obm_kernel(k_ref, v_ref, q_ref, o_ref, n_scr, z_scr):
    f32 = jnp.float32
    bf16 = jnp.bfloat16
    K = k_ref[0]
    V = v_ref[0]
    Q = q_ref[0]
    S, D = K.shape
    I = HIDDEN_SIZE

    iota = jax.lax.broadcasted_iota(jnp.int32, (S, D), 1)

    absk = jnp.abs(K)
    mk = jnp.max(absk, axis=1, keepdims=True)
    assign = jnp.min(jnp.where(absk == mk, iota, D), axis=1, keepdims=True)

    Kb = K.astype(bf16)
    Vb = V.astype(bf16)
    Qb = Q.astype(bf16)

    def slot_body(i, _):
        mask = assign == i
        vm = jnp.where(mask, Vb, bf16(0))
        km_f = jnp.where(mask, K, f32(0))
        M_i = jax.lax.dot_general(vm, Kb, (((0,), (0,)), ((), ())),
                                  preferred_element_type=f32)
        num_i = jax.lax.dot_general(Qb, M_i.astype(bf16), (((1,), (1,)), ((), ())),
                                    preferred_element_type=f32)
        z_i = jnp.sum(km_f, axis=0)
        n_scr[i] = num_i
        z_scr[pl.ds(i, 1), :] = z_i.reshape(1, D)
        return _

    jax.lax.fori_loop(0, I, slot_body, 0, unroll=4)

    Z = z_scr[:, :]
    den = jnp.sum(Q[:, None, :] * Z[None, :, :], axis=2) + EPS

    absq = jnp.abs(Q)
    remaining = absq
    m0 = jnp.max(remaining, axis=1, keepdims=True)
    topexp = jnp.zeros((S, D), f32)
    expsum = jnp.zeros((S, 1), f32)
    for _ in range(TOP_K):
        mt = jnp.max(remaining, axis=1, keepdims=True)
        ft = jnp.min(jnp.where(remaining == mt, iota, D), axis=1, keepdims=True)
        oh = iota == ft
        e = jnp.exp(mt - m0)
        topexp = topexp + jnp.where(oh, e, f32(0))
        expsum = expsum + e
        remaining = jnp.where(oh, -jnp.inf, remaining)

    P = topexp / (expsum * den)

    N = n_scr[...]
    Pt = P.T
    o_ref[0] = jnp.sum(Pt[:, :, None] * N, axis=0)


@jax.jit
def kernel(keys, values, queries):
    B, H, S, D = keys.shape
    ks = keys.reshape(B * H, S, D)
    vs = values.reshape(B * H, S, D)
    qs = queries.reshape(B * H, S, D)
    spec = pl.BlockSpec((1, S, D), lambda i: (i, 0, 0))
    out = pl.pallas_call(
        _obm_kernel,
        grid=(B * H,),
        in_specs=[spec, spec, spec],
        out_specs=spec,
        out_shape=jax.ShapeDtypeStruct((B * H, S, D), keys.dtype),
        scratch_shapes=[
            pltpu.VMEM((HIDDEN_SIZE, S, D), jnp.float32),
            pltpu.VMEM((HIDDEN_SIZE, D), jnp.float32),
        ],
    )(ks, vs, qs)
    return out.reshape(B, H, S, D)


# speedup vs baseline: 7.3671x; 3.0611x over previous
import jax
import jax.numpy as jnp
from jax.experimental import pallas as pl
from jax.experimental.pallas import tpu as pltpu

HIDDEN_SIZE = 64
TOP_K = 8
EPS = 1e-06


def _obm_kernel(k_ref, v_ref, q_ref, o_ref):
    f32 = jnp.float32
    bf16 = jnp.bfloat16
    K = k_ref[0]
    V = v_ref[0]
    Q = q_ref[0]
    S, D = K.shape
    I = HIDDEN_SIZE

    iota = jax.lax.broadcasted_iota(jnp.int32, (S, D), 1)

    absk = jnp.abs(K)
    mk = jnp.max(absk, axis=1, keepdims=True)
    assign = jnp.min(jnp.where(absk == mk, iota, D), axis=1, keepdims=True)

    Kb = K.astype(bf16)
    Vb = V.astype(bf16)
    Qb = Q.astype(bf16)

    iota_si = jax.lax.broadcasted_iota(jnp.int32, (S, I), 1)
    E2 = (iota_si == assign).astype(f32)
    E2T = E2.T

    W3 = E2T[:, :, None].astype(bf16) * Vb[None, :, :]
    M3 = jax.lax.dot_general(W3, Kb, (((1,), (0,)), ((), ())),
                             preferred_element_type=f32)

    num3 = jax.lax.dot_general(M3.astype(bf16), Qb, (((2,), (1,)), ((), ())),
                               preferred_element_type=f32)

    z3 = E2T[:, :, None] * K[None, :, :]
    Z = jnp.sum(z3, axis=1)
    den = jnp.sum(Q[:, None, :] * Z[None, :, :], axis=2) + EPS

    absq = jnp.abs(Q)
    remaining = absq
    m0 = jnp.max(remaining, axis=1, keepdims=True)
    topexp = jnp.zeros((S, D), f32)
    expsum = jnp.zeros((S, 1), f32)
    for _ in range(TOP_K):
        mt = jnp.max(remaining, axis=1, keepdims=True)
        ft = jnp.min(jnp.where(remaining == mt, iota, D), axis=1, keepdims=True)
        oh = iota == ft
        e = jnp.exp(mt - m0)
        topexp = topexp + jnp.where(oh, e, f32(0))
        expsum = expsum + e
        remaining = jnp.where(oh, -jnp.inf, remaining)

    P = topexp / (expsum * den)

    Pt = P.T
    out_t = jnp.sum(num3 * Pt[:, None, :], axis=0)
    o_ref[0] = out_t.T


@jax.jit
def kernel(keys, values, queries):
    B, H, S, D = keys.shape
    ks = keys.reshape(B * H, S, D)
    vs = values.reshape(B * H, S, D)
    qs = queries.reshape(B * H, S, D)
    spec = pl.BlockSpec((1, S, D), lambda i: (i, 0, 0))
    out = pl.pallas_call(
        _obm_kernel,
        grid=(B * H,),
        in_specs=[spec, spec, spec],
        out_specs=spec,
        out_shape=jax.ShapeDtypeStruct((B * H, S, D), keys.dtype),
    )(ks, vs, qs)
    return out.reshape(B, H, S, D)


# speedup vs baseline: 12.4804x; 1.6941x over previous
import jax
import jax.numpy as jnp
from jax.experimental import pallas as pl

HIDDEN_SIZE = 64
TOP_K = 8
EPS = 1e-06
BH_PER = 8


def _obm_kernel(k_ref, v_ref, q_ref, o_ref):
    f32 = jnp.float32
    bf16 = jnp.bfloat16
    G, S, D = k_ref.shape
    I = HIDDEN_SIZE
    Kf = k_ref[...].reshape(G * S, D)
    Vf = v_ref[...].reshape(G * S, D)
    Qf = q_ref[...].reshape(G * S, D)

    iota = jax.lax.broadcasted_iota(jnp.int32, (G * S, D), 1)

    absk = jnp.abs(Kf)
    mk = jnp.max(absk, axis=1, keepdims=True)
    assign = jnp.min(jnp.where(absk == mk, iota, D), axis=1, keepdims=True)
    E2 = (iota == assign).astype(f32)

    absq = jnp.abs(Qf)
    remaining = absq
    m0 = jnp.max(remaining, axis=1, keepdims=True)
    topexp = jnp.zeros((G * S, D), f32)
    expsum = jnp.zeros((G * S, 1), f32)
    for _ in range(TOP_K):
        mt = jnp.max(remaining, axis=1, keepdims=True)
        ft = jnp.min(jnp.where(remaining == mt, iota, D), axis=1, keepdims=True)
        oh = iota == ft
        e = jnp.exp(mt - m0)
        topexp = topexp + jnp.where(oh, e, f32(0))
        expsum = expsum + e
        remaining = jnp.where(oh, -jnp.inf, remaining)

    Kb = Kf.astype(bf16)
    Vb = Vf.astype(bf16)
    Qb = Qf.astype(bf16)

    for j in range(G):
        sl = slice(j * S, (j + 1) * S)
        K_j = Kf[sl]
        Kb_j = Kb[sl]
        Vb_j = Vb[sl]
        Qb_j = Qb[sl]
        Q_j = Qf[sl]
        E2T = E2[sl].T

        W3 = E2T[:, :, None].astype(bf16) * Vb_j[None, :, :]
        M3 = jax.lax.dot_general(W3, Kb_j, (((1,), (0,)), ((), ())),
                                 preferred_element_type=f32)
        num3 = jax.lax.dot_general(M3.astype(bf16), Qb_j,
                                   (((2,), (1,)), ((), ())),
                                   preferred_element_type=f32)

        z3 = E2T[:, :, None] * K_j[None, :, :]
        Z = jnp.sum(z3, axis=1)
        den = jnp.sum(Q_j[:, None, :] * Z[None, :, :], axis=2) + EPS

        P = topexp[sl] / (expsum[sl] * den)
        Pt = P.T
        out_t = jnp.sum(num3 * Pt[:, None, :], axis=0)
        o_ref[j] = out_t.T


@jax.jit
def kernel(keys, values, queries):
    B, H, S, D = keys.shape
    ks = keys.reshape(B * H, S, D)
    vs = values.reshape(B * H, S, D)
    qs = queries.reshape(B * H, S, D)
    spec = pl.BlockSpec((BH_PER, S, D), lambda i: (i, 0, 0))
    out = pl.pallas_call(
        _obm_kernel,
        grid=(B * H // BH_PER,),
        in_specs=[spec, spec, spec],
        out_specs=spec,
        out_shape=jax.ShapeDtypeStruct((B * H, S, D), keys.dtype),
    )(ks, vs, qs)
    return out.reshape(B, H, S, D)
